# Initial kernel scaffold; baseline (speedup 1.0000x reference)
import jax
import jax.numpy as jnp
from jax.experimental import pallas as pl


def kernel(x, edge_index, e, fR_W1, fR_b1, fR_W2, fR_b2, fO_W1, fO_b1, fO_W2, fO_b2):
    raise NotImplementedError("write your pallas kernel here")


# speedup vs baseline: 2.0397x; 2.0397x over previous
import dataclasses
import functools

import jax
import jax.numpy as jnp
from jax import lax
from jax.experimental import pallas as pl
from jax.experimental.pallas import tpu as pltpu
from jax.experimental.pallas import tpu_sc as plsc

N_NODES = 10000
N_EDGES = 320000
NODE_DIM = 128
EDGE_DIM = 16
HIDDEN = 128

GW = 128
N_EDGES_PAD = 323584
SC_CORES = 2
SC_SUBCORES = 16
N_NODES_PAD = 10240
ROWS_PT = N_NODES_PAD // SC_SUBCORES

_vmesh = plsc.VectorSubcoreMesh(core_axis_name="core", subcore_axis_name="subcore")

_sc_cp = pltpu.CompilerParams()
if "needs_layout_passes" in pltpu.CompilerParams.__dataclass_fields__:
    _sc_cp = dataclasses.replace(_sc_cp, needs_layout_passes=False)


N_WORKERS = SC_CORES * SC_SUBCORES
EPW = N_EDGES_PAD // N_WORKERS
NST = EPW // GW


def _sc_gather(xd, xs, dst_g, src_g):
    @functools.partial(
        pl.kernel,
        out_type=(
            jax.ShapeDtypeStruct((N_EDGES_PAD, HIDDEN), jnp.float32),
            jax.ShapeDtypeStruct((N_EDGES_PAD, HIDDEN), jnp.float32),
        ),
        mesh=_vmesh,
        scratch_types=[
            pltpu.VMEM((GW,), jnp.int32),
            pltpu.VMEM((GW,), jnp.int32),
            pltpu.VMEM((GW, HIDDEN), jnp.float32),
            pltpu.VMEM((GW, HIDDEN), jnp.float32),
            pltpu.SemaphoreType.DMA,
            pltpu.SemaphoreType.DMA,
        ],
    )
    def k(xd_hbm, xs_hbm, d_hbm, s_hbm, g1_hbm, g2_hbm, dv, sv, bufA, bufB, semA, semB):
        wid = lax.axis_index("subcore") * SC_CORES + lax.axis_index("core")
        base0 = wid * EPW

        @pl.loop(0, NST)
        def _(t):
            base = base0 + t * GW
            pltpu.sync_copy(d_hbm.at[pl.ds(base, GW)], dv)
            pltpu.sync_copy(s_hbm.at[pl.ds(base, GW)], sv)
            ca = pltpu.async_copy(xd_hbm.at[dv], bufA, semA)
            cb = pltpu.async_copy(xs_hbm.at[sv], bufB, semB)
            ca.wait()
            cb.wait()
            pltpu.sync_copy(bufA, g1_hbm.at[pl.ds(base, GW)])
            pltpu.sync_copy(bufB, g2_hbm.at[pl.ds(base, GW)])

    return k(xd, xs, dst_g, src_g)


NHALF = N_NODES_PAD // 2
EPP = N_EDGES_PAD // SC_SUBCORES
SCH = 256
NCH = EPP // SCH


def _sc_scatter(e_t, dst_s):
    @functools.partial(
        pl.kernel,
        out_type=jax.ShapeDtypeStruct(
            (SC_SUBCORES, SC_CORES, NHALF * EDGE_DIM), jnp.float32
        ),
        mesh=_vmesh,
        scratch_types=[
            pltpu.VMEM((SCH,), jnp.int32),
            pltpu.VMEM((EDGE_DIM, SCH), jnp.float32),
            pltpu.VMEM((NHALF * EDGE_DIM,), jnp.float32),
        ],
        compiler_params=_sc_cp,
    )
    def k(et_hbm, d_hbm, out_hbm, dv, ebt, acc):
        cid = lax.axis_index("core")
        sid = lax.axis_index("subcore")
        lo = cid * NHALF

        @pl.loop(0, NHALF * EDGE_DIM, step=16)
        def _(i):
            acc[pl.ds(i, 16)] = jnp.zeros((16,), jnp.float32)

        base0 = sid * EPP

        @pl.loop(0, NCH)
        def _(t):
            base = base0 + t * SCH
            pltpu.sync_copy(d_hbm.at[pl.ds(base, SCH)], dv)
            pltpu.sync_copy(et_hbm.at[:, pl.ds(base, SCH)], ebt)

            @pl.loop(0, SCH // 16)
            def _(g):
                off = g * 16
                dvv = dv[pl.ds(off, 16)]
                rows = dvv - lo
                mask = (rows >= 0) & (rows < NHALF)
                idxb = rows * EDGE_DIM
                for c in range(EDGE_DIM):
                    plsc.addupdate_scatter(
                        acc, [idxb + c], ebt[c, pl.ds(off, 16)], mask=mask
                    )

        pltpu.sync_copy(acc, out_hbm.at[sid, cid])

    return k(e_t, dst_s)


def _tc_pre_kernel(x_ref, wd_ref, ws_ref, xd_ref, xs_ref):
    xb = x_ref[...]
    xd_ref[...] = jnp.dot(xb, wd_ref[...], preferred_element_type=jnp.float32)
    xs_ref[...] = jnp.dot(xb, ws_ref[...], preferred_element_type=jnp.float32)


def _tc_pre(x, w_d, w_s):
    blk = 2000
    return pl.pallas_call(
        _tc_pre_kernel,
        grid=(N_NODES // blk,),
        in_specs=[
            pl.BlockSpec((blk, NODE_DIM), lambda i: (i, 0)),
            pl.BlockSpec((NODE_DIM, HIDDEN), lambda i: (0, 0)),
            pl.BlockSpec((NODE_DIM, HIDDEN), lambda i: (0, 0)),
        ],
        out_specs=[
            pl.BlockSpec((blk, HIDDEN), lambda i: (i, 0)),
            pl.BlockSpec((blk, HIDDEN), lambda i: (i, 0)),
        ],
        out_shape=[
            jax.ShapeDtypeStruct((N_NODES, HIDDEN), jnp.float32),
            jax.ShapeDtypeStruct((N_NODES, HIDDEN), jnp.float32),
        ],
    )(x, w_d, w_s)


def _tc_edge_kernel(
    g1_ref, g2_ref, e_ref, w1e_ref, b1_ref, w2_ref, b2_ref, out_ref, out_t_ref
):
    h = g1_ref[...] + g2_ref[...]
    h += jnp.dot(e_ref[...], w1e_ref[...], preferred_element_type=jnp.float32)
    h = jnp.maximum(h + b1_ref[...], 0.0)
    res = jnp.dot(h, w2_ref[...], preferred_element_type=jnp.float32) + b2_ref[...]
    out_ref[...] = res
    out_t_ref[...] = res.T


def _tc_edge(g1, g2, e, w1e, b1, w2, b2):
    blk = 2048
    return pl.pallas_call(
        _tc_edge_kernel,
        grid=(N_EDGES_PAD // blk,),
        in_specs=[
            pl.BlockSpec((blk, HIDDEN), lambda i: (i, 0)),
            pl.BlockSpec((blk, HIDDEN), lambda i: (i, 0)),
            pl.BlockSpec((blk, EDGE_DIM), lambda i: (i, 0)),
            pl.BlockSpec((EDGE_DIM, HIDDEN), lambda i: (0, 0)),
            pl.BlockSpec((1, HIDDEN), lambda i: (0, 0)),
            pl.BlockSpec((HIDDEN, EDGE_DIM), lambda i: (0, 0)),
            pl.BlockSpec((1, EDGE_DIM), lambda i: (0, 0)),
        ],
        out_specs=[
            pl.BlockSpec((blk, EDGE_DIM), lambda i: (i, 0)),
            pl.BlockSpec((EDGE_DIM, blk), lambda i: (0, i)),
        ],
        out_shape=[
            jax.ShapeDtypeStruct((N_EDGES_PAD, EDGE_DIM), jnp.float32),
            jax.ShapeDtypeStruct((EDGE_DIM, N_EDGES_PAD), jnp.float32),
        ],
    )(g1, g2, e, w1e, b1, w2, b2)


def _tc_node_kernel(x_ref, a_ref, w1x_ref, w1a_ref, b1_ref, w2_ref, b2_ref, out_ref):
    a = jnp.sum(a_ref[...], axis=0)
    h = jnp.dot(x_ref[...], w1x_ref[...], preferred_element_type=jnp.float32)
    h += jnp.dot(a, w1a_ref[...], preferred_element_type=jnp.float32)
    h = jnp.maximum(h + b1_ref[...], 0.0)
    out_ref[...] = (
        jnp.dot(h, w2_ref[...], preferred_element_type=jnp.float32) + b2_ref[...]
    )


def _tc_node(x, agg2, w1x, w1a, b1, w2, b2):
    blk = 1280
    return pl.pallas_call(
        _tc_node_kernel,
        grid=(N_NODES_PAD // blk,),
        in_specs=[
            pl.BlockSpec((blk, NODE_DIM), lambda i: (i, 0)),
            pl.BlockSpec((SC_SUBCORES, blk, EDGE_DIM), lambda i: (0, i, 0)),
            pl.BlockSpec((NODE_DIM, HIDDEN), lambda i: (0, 0)),
            pl.BlockSpec((EDGE_DIM, HIDDEN), lambda i: (0, 0)),
            pl.BlockSpec((1, HIDDEN), lambda i: (0, 0)),
            pl.BlockSpec((HIDDEN, NODE_DIM), lambda i: (0, 0)),
            pl.BlockSpec((1, NODE_DIM), lambda i: (0, 0)),
        ],
        out_specs=pl.BlockSpec((blk, NODE_DIM), lambda i: (i, 0)),
        out_shape=jax.ShapeDtypeStruct((N_NODES_PAD, NODE_DIM), jnp.float32),
    )(x, agg2, w1x, w1a, b1, w2, b2)


def kernel(x, edge_index, e, fR_W1, fR_b1, fR_W2, fR_b2, fO_W1, fO_b1, fO_W2, fO_b2):
    ei = edge_index.astype(jnp.int32)
    pad = N_EDGES_PAD - N_EDGES
    src2 = jnp.pad(ei[0], (0, pad))
    dst2 = jnp.pad(ei[1], (0, pad))
    dst2_s = jnp.pad(ei[1], (0, pad), constant_values=N_NODES_PAD - 8)
    e_pad = jnp.pad(e, ((0, pad), (0, 0)))

    xd, xs = _tc_pre(x, fR_W1[:NODE_DIM], fR_W1[NODE_DIM : 2 * NODE_DIM])
    g1, g2 = _sc_gather(xd, xs, dst2, src2)
    e_new_p, e_t = _tc_edge(
        g1, g2, e_pad,
        fR_W1[2 * NODE_DIM :],
        fR_b1.reshape(1, HIDDEN),
        fR_W2,
        fR_b2.reshape(1, EDGE_DIM),
    )
    e_new = e_new_p[:N_EDGES]
    parts = _sc_scatter(e_t, dst2_s).reshape(SC_SUBCORES, N_NODES_PAD, EDGE_DIM)
    x_pad = jnp.pad(x, ((0, N_NODES_PAD - N_NODES), (0, 0)))
    x_new = _tc_node(
        x_pad, parts,
        fO_W1[:NODE_DIM],
        fO_W1[NODE_DIM:],
        fO_b1.reshape(1, HIDDEN),
        fO_W2,
        fO_b2.reshape(1, NODE_DIM),
    )[:N_NODES]
    return (x_new, e_new)


# speedup vs baseline: 2.3728x; 1.1633x over previous
import dataclasses
import functools

import jax
import jax.numpy as jnp
from jax import lax
from jax.experimental import pallas as pl
from jax.experimental.pallas import tpu as pltpu
from jax.experimental.pallas import tpu_sc as plsc

N_NODES = 10000
N_EDGES = 320000
NODE_DIM = 128
EDGE_DIM = 16
HIDDEN = 128

GW = 128
N_EDGES_PAD = 323584
SC_CORES = 2
SC_SUBCORES = 16
N_NODES_PAD = 10240
ROWS_PT = N_NODES_PAD // SC_SUBCORES

_vmesh = plsc.VectorSubcoreMesh(core_axis_name="core", subcore_axis_name="subcore")

_sc_cp = pltpu.CompilerParams()
if "needs_layout_passes" in pltpu.CompilerParams.__dataclass_fields__:
    _sc_cp = dataclasses.replace(_sc_cp, needs_layout_passes=False)


N_WORKERS = SC_CORES * SC_SUBCORES
EPW = N_EDGES_PAD // N_WORKERS
NST = EPW // GW


def _sc_gather(xd, xs, dst_g, src_g):
    @functools.partial(
        pl.kernel,
        out_type=(
            jax.ShapeDtypeStruct((N_EDGES_PAD, HIDDEN), jnp.float32),
            jax.ShapeDtypeStruct((N_EDGES_PAD, HIDDEN), jnp.float32),
        ),
        mesh=_vmesh,
        scratch_types=[
            pltpu.VMEM((GW,), jnp.int32),
            pltpu.VMEM((GW,), jnp.int32),
            pltpu.VMEM((GW,), jnp.int32),
            pltpu.VMEM((GW,), jnp.int32),
            pltpu.VMEM((GW, HIDDEN), jnp.float32),
            pltpu.VMEM((GW, HIDDEN), jnp.float32),
            pltpu.VMEM((GW, HIDDEN), jnp.float32),
            pltpu.VMEM((GW, HIDDEN), jnp.float32),
            pltpu.SemaphoreType.DMA,
            pltpu.SemaphoreType.DMA,
            pltpu.SemaphoreType.DMA,
            pltpu.SemaphoreType.DMA,
            pltpu.SemaphoreType.DMA,
            pltpu.SemaphoreType.DMA,
            pltpu.SemaphoreType.DMA,
            pltpu.SemaphoreType.DMA,
        ],
    )
    def k(xd_hbm, xs_hbm, d_hbm, s_hbm, g1_hbm, g2_hbm,
          dv0, dv1, sv0, sv1, A0, A1, B0, B1,
          gsA0, gsA1, gsB0, gsB1, wsA0, wsA1, wsB0, wsB1):
        wid = lax.axis_index("subcore") * SC_CORES + lax.axis_index("core")
        base0 = wid * EPW
        dv = (dv0, dv1)
        sv = (sv0, sv1)
        A = (A0, A1)
        B = (B0, B1)
        gsA = (gsA0, gsA1)
        gsB = (gsB0, gsB1)
        wsA = (wsA0, wsA1)
        wsB = (wsB0, wsB1)

        def issue(s, t):
            base = base0 + t * GW
            pltpu.sync_copy(d_hbm.at[pl.ds(base, GW)], dv[s])
            pltpu.sync_copy(s_hbm.at[pl.ds(base, GW)], sv[s])
            pltpu.async_copy(xd_hbm.at[dv[s]], A[s], gsA[s])
            pltpu.async_copy(xs_hbm.at[sv[s]], B[s], gsB[s])

        def wait_g(s):
            pltpu.make_async_copy(xd_hbm.at[dv[s]], A[s], gsA[s]).wait()
            pltpu.make_async_copy(xs_hbm.at[sv[s]], B[s], gsB[s]).wait()

        def issue_wb(s, t):
            base = base0 + t * GW
            pltpu.async_copy(A[s], g1_hbm.at[pl.ds(base, GW)], wsA[s])
            pltpu.async_copy(B[s], g2_hbm.at[pl.ds(base, GW)], wsB[s])

        def wait_wb(s):
            pltpu.make_async_copy(A[s], g1_hbm.at[pl.ds(base0, GW)], wsA[s]).wait()
            pltpu.make_async_copy(B[s], g2_hbm.at[pl.ds(base0, GW)], wsB[s]).wait()

        issue(0, 0)

        @pl.loop(0, NST // 2)
        def _(i):
            t0 = 2 * i

            @pl.when(i > 0)
            def _():
                wait_wb(1)

            issue(1, t0 + 1)
            wait_g(0)
            issue_wb(0, t0)
            wait_g(1)
            issue_wb(1, t0 + 1)
            wait_wb(0)
            issue(0, t0 + 2)

        wait_g(0)
        issue_wb(0, NST - 1)
        wait_wb(1)
        wait_wb(0)

    return k(xd, xs, dst_g, src_g)


NHALF = N_NODES_PAD // 2
EPP = N_EDGES_PAD // SC_SUBCORES
SCH = 256
NCH = EPP // SCH


def _sc_scatter(e_t, dst_s):
    @functools.partial(
        pl.kernel,
        out_type=jax.ShapeDtypeStruct(
            (SC_SUBCORES, SC_CORES, NHALF * EDGE_DIM), jnp.float32
        ),
        mesh=_vmesh,
        scratch_types=[
            pltpu.VMEM((SCH,), jnp.int32),
            pltpu.VMEM((SCH,), jnp.int32),
            pltpu.VMEM((EDGE_DIM, SCH), jnp.float32),
            pltpu.VMEM((EDGE_DIM, SCH), jnp.float32),
            pltpu.VMEM((NHALF * EDGE_DIM,), jnp.float32),
            pltpu.SemaphoreType.DMA,
            pltpu.SemaphoreType.DMA,
            pltpu.SemaphoreType.DMA,
            pltpu.SemaphoreType.DMA,
        ],
        compiler_params=_sc_cp,
    )
    def k(et_hbm, d_hbm, out_hbm, dv0, dv1, eb0, eb1, acc, dm0, dm1, em0, em1):
        cid = lax.axis_index("core")
        sid = lax.axis_index("subcore")
        lo = cid * NHALF
        dvs = (dv0, dv1)
        ebs = (eb0, eb1)
        dsm = (dm0, dm1)
        esm = (em0, em1)

        @pl.loop(0, NHALF * EDGE_DIM, step=16)
        def _(i):
            acc[pl.ds(i, 16)] = jnp.zeros((16,), jnp.float32)

        base0 = sid * EPP

        def issue(s, t):
            base = base0 + t * SCH
            pltpu.async_copy(d_hbm.at[pl.ds(base, SCH)], dvs[s], dsm[s])
            pltpu.async_copy(et_hbm.at[:, pl.ds(base, SCH)], ebs[s], esm[s])

        def wait_l(s):
            pltpu.make_async_copy(d_hbm.at[pl.ds(base0, SCH)], dvs[s], dsm[s]).wait()
            pltpu.make_async_copy(
                et_hbm.at[:, pl.ds(base0, SCH)], ebs[s], esm[s]
            ).wait()

        def process(s):
            @pl.loop(0, SCH // 16)
            def _(g):
                off = g * 16
                dvv = dvs[s][pl.ds(off, 16)]
                rows = dvv - lo
                mask = (rows >= 0) & (rows < NHALF)
                idxb = rows * EDGE_DIM
                for c in range(EDGE_DIM):
                    plsc.addupdate_scatter(
                        acc, [idxb + c], ebs[s][c, pl.ds(off, 16)], mask=mask
                    )

        issue(0, 0)

        @pl.loop(0, NCH // 2)
        def _(i):
            t0 = 2 * i
            issue(1, t0 + 1)
            wait_l(0)
            process(0)
            issue(0, t0 + 2)
            wait_l(1)
            process(1)

        wait_l(0)
        process(0)
        pltpu.sync_copy(acc, out_hbm.at[sid, cid])

    return k(e_t, dst_s)


def _tc_pre_kernel(x_ref, wd_ref, ws_ref, xd_ref, xs_ref):
    xb = x_ref[...]
    xd_ref[...] = jnp.dot(xb, wd_ref[...], preferred_element_type=jnp.float32)
    xs_ref[...] = jnp.dot(xb, ws_ref[...], preferred_element_type=jnp.float32)


def _tc_pre(x, w_d, w_s):
    blk = 2000
    return pl.pallas_call(
        _tc_pre_kernel,
        grid=(N_NODES // blk,),
        in_specs=[
            pl.BlockSpec((blk, NODE_DIM), lambda i: (i, 0)),
            pl.BlockSpec((NODE_DIM, HIDDEN), lambda i: (0, 0)),
            pl.BlockSpec((NODE_DIM, HIDDEN), lambda i: (0, 0)),
        ],
        out_specs=[
            pl.BlockSpec((blk, HIDDEN), lambda i: (i, 0)),
            pl.BlockSpec((blk, HIDDEN), lambda i: (i, 0)),
        ],
        out_shape=[
            jax.ShapeDtypeStruct((N_NODES, HIDDEN), jnp.float32),
            jax.ShapeDtypeStruct((N_NODES, HIDDEN), jnp.float32),
        ],
    )(x, w_d, w_s)


def _tc_edge_kernel(
    g1_ref, g2_ref, e_ref, w1e_ref, b1_ref, w2_ref, b2_ref, out_ref, out_t_ref
):
    h = g1_ref[...] + g2_ref[...]
    h += jnp.dot(e_ref[...], w1e_ref[...], preferred_element_type=jnp.float32)
    h = jnp.maximum(h + b1_ref[...], 0.0)
    res = jnp.dot(h, w2_ref[...], preferred_element_type=jnp.float32) + b2_ref[...]
    out_ref[...] = res
    out_t_ref[...] = res.T


def _tc_edge(g1, g2, e, w1e, b1, w2, b2):
    blk = 2048
    return pl.pallas_call(
        _tc_edge_kernel,
        grid=(N_EDGES_PAD // blk,),
        in_specs=[
            pl.BlockSpec((blk, HIDDEN), lambda i: (i, 0)),
            pl.BlockSpec((blk, HIDDEN), lambda i: (i, 0)),
            pl.BlockSpec((blk, EDGE_DIM), lambda i: (i, 0)),
            pl.BlockSpec((EDGE_DIM, HIDDEN), lambda i: (0, 0)),
            pl.BlockSpec((1, HIDDEN), lambda i: (0, 0)),
            pl.BlockSpec((HIDDEN, EDGE_DIM), lambda i: (0, 0)),
            pl.BlockSpec((1, EDGE_DIM), lambda i: (0, 0)),
        ],
        out_specs=[
            pl.BlockSpec((blk, EDGE_DIM), lambda i: (i, 0)),
            pl.BlockSpec((EDGE_DIM, blk), lambda i: (0, i)),
        ],
        out_shape=[
            jax.ShapeDtypeStruct((N_EDGES_PAD, EDGE_DIM), jnp.float32),
            jax.ShapeDtypeStruct((EDGE_DIM, N_EDGES_PAD), jnp.float32),
        ],
    )(g1, g2, e, w1e, b1, w2, b2)


def _tc_node_kernel(x_ref, a_ref, w1x_ref, w1a_ref, b1_ref, w2_ref, b2_ref, out_ref):
    a = jnp.sum(a_ref[...], axis=0)
    h = jnp.dot(x_ref[...], w1x_ref[...], preferred_element_type=jnp.float32)
    h += jnp.dot(a, w1a_ref[...], preferred_element_type=jnp.float32)
    h = jnp.maximum(h + b1_ref[...], 0.0)
    out_ref[...] = (
        jnp.dot(h, w2_ref[...], preferred_element_type=jnp.float32) + b2_ref[...]
    )


def _tc_node(x, agg2, w1x, w1a, b1, w2, b2):
    blk = 1280
    return pl.pallas_call(
        _tc_node_kernel,
        grid=(N_NODES_PAD // blk,),
        in_specs=[
            pl.BlockSpec((blk, NODE_DIM), lambda i: (i, 0)),
            pl.BlockSpec((SC_SUBCORES, blk, EDGE_DIM), lambda i: (0, i, 0)),
            pl.BlockSpec((NODE_DIM, HIDDEN), lambda i: (0, 0)),
            pl.BlockSpec((EDGE_DIM, HIDDEN), lambda i: (0, 0)),
            pl.BlockSpec((1, HIDDEN), lambda i: (0, 0)),
            pl.BlockSpec((HIDDEN, NODE_DIM), lambda i: (0, 0)),
            pl.BlockSpec((1, NODE_DIM), lambda i: (0, 0)),
        ],
        out_specs=pl.BlockSpec((blk, NODE_DIM), lambda i: (i, 0)),
        out_shape=jax.ShapeDtypeStruct((N_NODES_PAD, NODE_DIM), jnp.float32),
    )(x, agg2, w1x, w1a, b1, w2, b2)


def kernel(x, edge_index, e, fR_W1, fR_b1, fR_W2, fR_b2, fO_W1, fO_b1, fO_W2, fO_b2):
    ei = edge_index.astype(jnp.int32)
    pad = N_EDGES_PAD - N_EDGES
    src2 = jnp.pad(ei[0], (0, pad))
    dst2 = jnp.pad(ei[1], (0, pad))
    dst2_s = jnp.pad(ei[1], (0, pad), constant_values=N_NODES_PAD - 8)
    e_pad = jnp.pad(e, ((0, pad), (0, 0)))

    xd, xs = _tc_pre(x, fR_W1[:NODE_DIM], fR_W1[NODE_DIM : 2 * NODE_DIM])
    g1, g2 = _sc_gather(xd, xs, dst2, src2)
    e_new_p, e_t = _tc_edge(
        g1, g2, e_pad,
        fR_W1[2 * NODE_DIM :],
        fR_b1.reshape(1, HIDDEN),
        fR_W2,
        fR_b2.reshape(1, EDGE_DIM),
    )
    e_new = e_new_p[:N_EDGES]
    parts = _sc_scatter(e_t, dst2_s).reshape(SC_SUBCORES, N_NODES_PAD, EDGE_DIM)
    x_pad = jnp.pad(x, ((0, N_NODES_PAD - N_NODES), (0, 0)))
    x_new = _tc_node(
        x_pad, parts,
        fO_W1[:NODE_DIM],
        fO_W1[NODE_DIM:],
        fO_b1.reshape(1, HIDDEN),
        fO_W2,
        fO_b2.reshape(1, NODE_DIM),
    )[:N_NODES]
    return (x_new, e_new)


# speedup vs baseline: 2.5050x; 1.0557x over previous
import dataclasses
import functools

import jax
import jax.numpy as jnp
from jax import lax
from jax.experimental import pallas as pl
from jax.experimental.pallas import tpu as pltpu
from jax.experimental.pallas import tpu_sc as plsc

N_NODES = 10000
N_EDGES = 320000
NODE_DIM = 128
EDGE_DIM = 16
HIDDEN = 128

GW = 128
N_EDGES_PAD = 323584
SC_CORES = 2
SC_SUBCORES = 16
N_NODES_PAD = 10240
ROWS_PT = N_NODES_PAD // SC_SUBCORES

_vmesh = plsc.VectorSubcoreMesh(core_axis_name="core", subcore_axis_name="subcore")

_sc_cp = pltpu.CompilerParams()
if "needs_layout_passes" in pltpu.CompilerParams.__dataclass_fields__:
    _sc_cp = dataclasses.replace(_sc_cp, needs_layout_passes=False)


N_WORKERS = SC_CORES * SC_SUBCORES
EPW = N_EDGES_PAD // N_WORKERS
NST = EPW // GW


def _sc_gather(xd, xs, dst_g, src_g):
    @functools.partial(
        pl.kernel,
        out_type=jax.ShapeDtypeStruct((N_EDGES_PAD, HIDDEN), jnp.float32),
        mesh=_vmesh,
        scratch_types=[
            pltpu.VMEM((EPW,), jnp.int32),
            pltpu.VMEM((EPW,), jnp.int32),
            pltpu.VMEM((GW, HIDDEN), jnp.float32),
            pltpu.VMEM((GW, HIDDEN), jnp.float32),
            pltpu.VMEM((GW, HIDDEN), jnp.float32),
            pltpu.VMEM((GW, HIDDEN), jnp.float32),
            pltpu.SemaphoreType.DMA,
            pltpu.SemaphoreType.DMA,
            pltpu.SemaphoreType.DMA,
            pltpu.SemaphoreType.DMA,
            pltpu.SemaphoreType.DMA,
            pltpu.SemaphoreType.DMA,
        ],
        compiler_params=_sc_cp,
    )
    def k(xd_hbm, xs_hbm, d_hbm, s_hbm, g_hbm,
          dva, sva, A0, A1, B0, B1, gsA0, gsA1, gsB0, gsB1, wsA0, wsA1):
        wid = lax.axis_index("subcore") * SC_CORES + lax.axis_index("core")
        base0 = wid * EPW
        A = (A0, A1)
        B = (B0, B1)
        gsA = (gsA0, gsA1)
        gsB = (gsB0, gsB1)
        wsA = (wsA0, wsA1)

        pltpu.sync_copy(d_hbm.at[pl.ds(base0, EPW)], dva)
        pltpu.sync_copy(s_hbm.at[pl.ds(base0, EPW)], sva)

        def issue(s, t):
            off = t * GW
            pltpu.async_copy(xd_hbm.at[dva.at[pl.ds(off, GW)]], A[s], gsA[s])
            pltpu.async_copy(xs_hbm.at[sva.at[pl.ds(off, GW)]], B[s], gsB[s])

        def wait_g(s):
            pltpu.make_async_copy(xd_hbm.at[dva.at[pl.ds(0, GW)]], A[s], gsA[s]).wait()
            pltpu.make_async_copy(xs_hbm.at[sva.at[pl.ds(0, GW)]], B[s], gsB[s]).wait()

        def add(s):
            @pl.loop(0, GW)
            def _(r):
                for j in range(HIDDEN // 16):
                    plsc.addupdate(
                        A[s].at[r, pl.ds(j * 16, 16)], B[s][r, pl.ds(j * 16, 16)]
                    )

        def issue_wb(s, t):
            base = base0 + t * GW
            pltpu.async_copy(A[s], g_hbm.at[pl.ds(base, GW)], wsA[s])

        def wait_wb(s):
            pltpu.make_async_copy(A[s], g_hbm.at[pl.ds(base0, GW)], wsA[s]).wait()

        issue(0, 0)

        @pl.loop(0, NST // 2)
        def _(i):
            t0 = 2 * i

            @pl.when(i > 0)
            def _():
                wait_wb(1)

            issue(1, t0 + 1)
            wait_g(0)
            add(0)
            issue_wb(0, t0)
            wait_g(1)
            add(1)
            issue_wb(1, t0 + 1)
            wait_wb(0)
            issue(0, t0 + 2)

        wait_g(0)
        add(0)
        issue_wb(0, NST - 1)
        wait_wb(1)
        wait_wb(0)

    return k(xd, xs, dst_g, src_g)


NHALF = N_NODES_PAD // 2
EPP = N_EDGES_PAD // SC_SUBCORES
SCH = 256
NCH = EPP // SCH


def _sc_scatter(e_t, dst_s):
    @functools.partial(
        pl.kernel,
        out_type=jax.ShapeDtypeStruct(
            (SC_SUBCORES, SC_CORES, NHALF * EDGE_DIM), jnp.float32
        ),
        mesh=_vmesh,
        scratch_types=[
            pltpu.VMEM((SCH,), jnp.int32),
            pltpu.VMEM((SCH,), jnp.int32),
            pltpu.VMEM((EDGE_DIM, SCH), jnp.float32),
            pltpu.VMEM((EDGE_DIM, SCH), jnp.float32),
            pltpu.VMEM((NHALF * EDGE_DIM,), jnp.float32),
            pltpu.SemaphoreType.DMA,
            pltpu.SemaphoreType.DMA,
            pltpu.SemaphoreType.DMA,
            pltpu.SemaphoreType.DMA,
        ],
        compiler_params=_sc_cp,
    )
    def k(et_hbm, d_hbm, out_hbm, dv0, dv1, eb0, eb1, acc, dm0, dm1, em0, em1):
        cid = lax.axis_index("core")
        sid = lax.axis_index("subcore")
        lo = cid * NHALF
        dvs = (dv0, dv1)
        ebs = (eb0, eb1)
        dsm = (dm0, dm1)
        esm = (em0, em1)

        @pl.loop(0, NHALF * EDGE_DIM, step=16)
        def _(i):
            acc[pl.ds(i, 16)] = jnp.zeros((16,), jnp.float32)

        base0 = sid * EPP

        def issue(s, t):
            base = base0 + t * SCH
            pltpu.async_copy(d_hbm.at[pl.ds(base, SCH)], dvs[s], dsm[s])
            pltpu.async_copy(et_hbm.at[:, pl.ds(base, SCH)], ebs[s], esm[s])

        def wait_l(s):
            pltpu.make_async_copy(d_hbm.at[pl.ds(base0, SCH)], dvs[s], dsm[s]).wait()
            pltpu.make_async_copy(
                et_hbm.at[:, pl.ds(base0, SCH)], ebs[s], esm[s]
            ).wait()

        def process(s):
            @pl.loop(0, SCH // 16)
            def _(g):
                off = g * 16
                dvv = dvs[s][pl.ds(off, 16)]
                rows = dvv - lo
                mask = (rows >= 0) & (rows < NHALF)
                idxb = rows * EDGE_DIM
                for c in range(EDGE_DIM):
                    plsc.addupdate_scatter(
                        acc, [idxb + c], ebs[s][c, pl.ds(off, 16)], mask=mask
                    )

        issue(0, 0)

        @pl.loop(0, NCH // 2)
        def _(i):
            t0 = 2 * i
            issue(1, t0 + 1)
            wait_l(0)
            process(0)
            issue(0, t0 + 2)
            wait_l(1)
            process(1)

        wait_l(0)
        process(0)
        pltpu.sync_copy(acc, out_hbm.at[sid, cid])

    return k(e_t, dst_s)


def _tc_pre_kernel(x_ref, wd_ref, ws_ref, xd_ref, xs_ref):
    xb = x_ref[...]
    xd_ref[...] = jnp.dot(xb, wd_ref[...], preferred_element_type=jnp.float32)
    xs_ref[...] = jnp.dot(xb, ws_ref[...], preferred_element_type=jnp.float32)


def _tc_pre(x, w_d, w_s):
    blk = 2000
    return pl.pallas_call(
        _tc_pre_kernel,
        grid=(N_NODES // blk,),
        in_specs=[
            pl.BlockSpec((blk, NODE_DIM), lambda i: (i, 0)),
            pl.BlockSpec((NODE_DIM, HIDDEN), lambda i: (0, 0)),
            pl.BlockSpec((NODE_DIM, HIDDEN), lambda i: (0, 0)),
        ],
        out_specs=[
            pl.BlockSpec((blk, HIDDEN), lambda i: (i, 0)),
            pl.BlockSpec((blk, HIDDEN), lambda i: (i, 0)),
        ],
        out_shape=[
            jax.ShapeDtypeStruct((N_NODES, HIDDEN), jnp.float32),
            jax.ShapeDtypeStruct((N_NODES, HIDDEN), jnp.float32),
        ],
    )(x, w_d, w_s)


def _tc_edge_kernel(
    g_ref, e_ref, w1e_ref, b1_ref, w2_ref, b2_ref, out_ref, out_t_ref
):
    h = g_ref[...]
    h += jnp.dot(e_ref[...], w1e_ref[...], preferred_element_type=jnp.float32)
    h = jnp.maximum(h + b1_ref[...], 0.0)
    res = jnp.dot(h, w2_ref[...], preferred_element_type=jnp.float32) + b2_ref[...]
    out_ref[...] = res
    out_t_ref[...] = res.T


def _tc_edge(g, e, w1e, b1, w2, b2):
    blk = 2048
    return pl.pallas_call(
        _tc_edge_kernel,
        grid=(N_EDGES_PAD // blk,),
        in_specs=[
            pl.BlockSpec((blk, HIDDEN), lambda i: (i, 0)),
            pl.BlockSpec((blk, EDGE_DIM), lambda i: (i, 0)),
            pl.BlockSpec((EDGE_DIM, HIDDEN), lambda i: (0, 0)),
            pl.BlockSpec((1, HIDDEN), lambda i: (0, 0)),
            pl.BlockSpec((HIDDEN, EDGE_DIM), lambda i: (0, 0)),
            pl.BlockSpec((1, EDGE_DIM), lambda i: (0, 0)),
        ],
        out_specs=[
            pl.BlockSpec((blk, EDGE_DIM), lambda i: (i, 0)),
            pl.BlockSpec((EDGE_DIM, blk), lambda i: (0, i)),
        ],
        out_shape=[
            jax.ShapeDtypeStruct((N_EDGES_PAD, EDGE_DIM), jnp.float32),
            jax.ShapeDtypeStruct((EDGE_DIM, N_EDGES_PAD), jnp.float32),
        ],
    )(g, e, w1e, b1, w2, b2)


def _tc_node_kernel(x_ref, a_ref, w1x_ref, w1a_ref, b1_ref, w2_ref, b2_ref, out_ref):
    a = jnp.sum(a_ref[...], axis=0)
    h = jnp.dot(x_ref[...], w1x_ref[...], preferred_element_type=jnp.float32)
    h += jnp.dot(a, w1a_ref[...], preferred_element_type=jnp.float32)
    h = jnp.maximum(h + b1_ref[...], 0.0)
    out_ref[...] = (
        jnp.dot(h, w2_ref[...], preferred_element_type=jnp.float32) + b2_ref[...]
    )


def _tc_node(x, agg2, w1x, w1a, b1, w2, b2):
    blk = 1280
    return pl.pallas_call(
        _tc_node_kernel,
        grid=(N_NODES_PAD // blk,),
        in_specs=[
            pl.BlockSpec((blk, NODE_DIM), lambda i: (i, 0)),
            pl.BlockSpec((SC_SUBCORES, blk, EDGE_DIM), lambda i: (0, i, 0)),
            pl.BlockSpec((NODE_DIM, HIDDEN), lambda i: (0, 0)),
            pl.BlockSpec((EDGE_DIM, HIDDEN), lambda i: (0, 0)),
            pl.BlockSpec((1, HIDDEN), lambda i: (0, 0)),
            pl.BlockSpec((HIDDEN, NODE_DIM), lambda i: (0, 0)),
            pl.BlockSpec((1, NODE_DIM), lambda i: (0, 0)),
        ],
        out_specs=pl.BlockSpec((blk, NODE_DIM), lambda i: (i, 0)),
        out_shape=jax.ShapeDtypeStruct((N_NODES_PAD, NODE_DIM), jnp.float32),
    )(x, agg2, w1x, w1a, b1, w2, b2)


def kernel(x, edge_index, e, fR_W1, fR_b1, fR_W2, fR_b2, fO_W1, fO_b1, fO_W2, fO_b2):
    ei = edge_index.astype(jnp.int32)
    pad = N_EDGES_PAD - N_EDGES
    src2 = jnp.pad(ei[0], (0, pad))
    dst2 = jnp.pad(ei[1], (0, pad))
    dst2_s = jnp.pad(ei[1], (0, pad), constant_values=N_NODES_PAD - 8)
    e_pad = jnp.pad(e, ((0, pad), (0, 0)))

    xd, xs = _tc_pre(x, fR_W1[:NODE_DIM], fR_W1[NODE_DIM : 2 * NODE_DIM])
    g = _sc_gather(xd, xs, dst2, src2)
    e_new_p, e_t = _tc_edge(
        g, e_pad,
        fR_W1[2 * NODE_DIM :],
        fR_b1.reshape(1, HIDDEN),
        fR_W2,
        fR_b2.reshape(1, EDGE_DIM),
    )
    e_new = e_new_p[:N_EDGES]
    parts = _sc_scatter(e_t, dst2_s).reshape(SC_SUBCORES, N_NODES_PAD, EDGE_DIM)
    x_pad = jnp.pad(x, ((0, N_NODES_PAD - N_NODES), (0, 0)))
    x_new = _tc_node(
        x_pad, parts,
        fO_W1[:NODE_DIM],
        fO_W1[NODE_DIM:],
        fO_b1.reshape(1, HIDDEN),
        fO_W2,
        fO_b2.reshape(1, NODE_DIM),
    )[:N_NODES]
    return (x_new, e_new)


# speedup vs baseline: 2.7348x; 1.0918x over previous
import dataclasses
import functools

import jax
import jax.numpy as jnp
from jax import lax
from jax.experimental import pallas as pl
from jax.experimental.pallas import tpu as pltpu
from jax.experimental.pallas import tpu_sc as plsc

N_NODES = 10000
N_EDGES = 320000
NODE_DIM = 128
EDGE_DIM = 16
HIDDEN = 128

GW = 128
N_EDGES_PAD = 323584
SC_CORES = 2
SC_SUBCORES = 16
N_NODES_PAD = 10240
ROWS_PT = N_NODES_PAD // SC_SUBCORES

_vmesh = plsc.VectorSubcoreMesh(core_axis_name="core", subcore_axis_name="subcore")

_sc_cp = pltpu.CompilerParams()
if "needs_layout_passes" in pltpu.CompilerParams.__dataclass_fields__:
    _sc_cp = dataclasses.replace(_sc_cp, needs_layout_passes=False)


N_WORKERS = SC_CORES * SC_SUBCORES
EPW = N_EDGES_PAD // N_WORKERS
NST = EPW // GW


def _sc_gather(xd, xs, dst_g, src_g):
    @functools.partial(
        pl.kernel,
        out_type=jax.ShapeDtypeStruct((N_EDGES_PAD, HIDDEN), jnp.float32),
        mesh=_vmesh,
        scratch_types=[
            pltpu.VMEM((EPW,), jnp.int32),
            pltpu.VMEM((EPW,), jnp.int32),
            pltpu.VMEM((GW, HIDDEN), jnp.float32),
            pltpu.VMEM((GW, HIDDEN), jnp.float32),
            pltpu.VMEM((GW, HIDDEN), jnp.float32),
            pltpu.VMEM((GW, HIDDEN), jnp.float32),
            pltpu.SemaphoreType.DMA,
            pltpu.SemaphoreType.DMA,
            pltpu.SemaphoreType.DMA,
            pltpu.SemaphoreType.DMA,
            pltpu.SemaphoreType.DMA,
            pltpu.SemaphoreType.DMA,
        ],
        compiler_params=_sc_cp,
    )
    def k(xd_hbm, xs_hbm, d_hbm, s_hbm, g_hbm,
          dva, sva, A0, A1, B0, B1, gsA0, gsA1, gsB0, gsB1, wsA0, wsA1):
        wid = lax.axis_index("subcore") * SC_CORES + lax.axis_index("core")
        base0 = wid * EPW
        A = (A0, A1)
        B = (B0, B1)
        gsA = (gsA0, gsA1)
        gsB = (gsB0, gsB1)
        wsA = (wsA0, wsA1)

        pltpu.sync_copy(d_hbm.at[pl.ds(base0, EPW)], dva)
        pltpu.sync_copy(s_hbm.at[pl.ds(base0, EPW)], sva)

        def issue(s, t):
            off = t * GW
            pltpu.async_copy(xd_hbm.at[dva.at[pl.ds(off, GW)]], A[s], gsA[s])
            pltpu.async_copy(xs_hbm.at[sva.at[pl.ds(off, GW)]], B[s], gsB[s])

        def wait_g(s):
            pltpu.make_async_copy(xd_hbm.at[dva.at[pl.ds(0, GW)]], A[s], gsA[s]).wait()
            pltpu.make_async_copy(xs_hbm.at[sva.at[pl.ds(0, GW)]], B[s], gsB[s]).wait()

        def add(s):
            @pl.loop(0, GW)
            def _(r):
                for j in range(HIDDEN // 16):
                    plsc.addupdate(
                        A[s].at[r, pl.ds(j * 16, 16)], B[s][r, pl.ds(j * 16, 16)]
                    )

        def issue_wb(s, t):
            base = base0 + t * GW
            pltpu.async_copy(A[s], g_hbm.at[pl.ds(base, GW)], wsA[s])

        def wait_wb(s):
            pltpu.make_async_copy(A[s], g_hbm.at[pl.ds(base0, GW)], wsA[s]).wait()

        issue(0, 0)

        @pl.loop(0, NST // 2)
        def _(i):
            t0 = 2 * i

            @pl.when(i > 0)
            def _():
                wait_wb(1)

            issue(1, t0 + 1)
            wait_g(0)
            add(0)
            issue_wb(0, t0)
            wait_g(1)
            add(1)
            issue_wb(1, t0 + 1)
            wait_wb(0)
            issue(0, t0 + 2)

        wait_g(0)
        add(0)
        issue_wb(0, NST - 1)
        wait_wb(1)
        wait_wb(0)

    return k(xd, xs, dst_g, src_g)


NHALF = N_NODES_PAD // 2
EPP = N_EDGES_PAD // SC_SUBCORES
SCH = 256
NCH = EPP // SCH


def _sc_scatter(e_t, dst_s):
    @functools.partial(
        pl.kernel,
        out_type=jax.ShapeDtypeStruct(
            (SC_SUBCORES, SC_CORES, NHALF * EDGE_DIM), jnp.float32
        ),
        mesh=_vmesh,
        scratch_types=[
            pltpu.VMEM((SCH,), jnp.int32),
            pltpu.VMEM((SCH,), jnp.int32),
            pltpu.VMEM((EDGE_DIM, SCH), jnp.float32),
            pltpu.VMEM((EDGE_DIM, SCH), jnp.float32),
            pltpu.VMEM((NHALF * EDGE_DIM,), jnp.float32),
            pltpu.SemaphoreType.DMA,
            pltpu.SemaphoreType.DMA,
            pltpu.SemaphoreType.DMA,
            pltpu.SemaphoreType.DMA,
        ],
        compiler_params=_sc_cp,
    )
    def k(et_hbm, d_hbm, out_hbm, dv0, dv1, eb0, eb1, acc, dm0, dm1, em0, em1):
        cid = lax.axis_index("core")
        sid = lax.axis_index("subcore")
        lo = cid * NHALF
        dvs = (dv0, dv1)
        ebs = (eb0, eb1)
        dsm = (dm0, dm1)
        esm = (em0, em1)

        @pl.loop(0, NHALF * EDGE_DIM, step=16)
        def _(i):
            acc[pl.ds(i, 16)] = jnp.zeros((16,), jnp.float32)

        base0 = sid * EPP

        def issue(s, t):
            base = base0 + t * SCH
            pltpu.async_copy(d_hbm.at[pl.ds(base, SCH)], dvs[s], dsm[s])
            pltpu.async_copy(et_hbm.at[:, pl.ds(base, SCH)], ebs[s], esm[s])

        def wait_l(s):
            pltpu.make_async_copy(d_hbm.at[pl.ds(base0, SCH)], dvs[s], dsm[s]).wait()
            pltpu.make_async_copy(
                et_hbm.at[:, pl.ds(base0, SCH)], ebs[s], esm[s]
            ).wait()

        def process(s):
            @pl.loop(0, SCH // 16)
            def _(g):
                off = g * 16
                dvv = dvs[s][pl.ds(off, 16)]
                rows = dvv - lo
                mask = (rows >= 0) & (rows < NHALF)
                idxb = rows * EDGE_DIM
                for c in range(EDGE_DIM):
                    plsc.addupdate_scatter(
                        acc, [idxb + c], ebs[s][c, pl.ds(off, 16)], mask=mask
                    )

        issue(0, 0)

        @pl.loop(0, NCH // 2)
        def _(i):
            t0 = 2 * i
            issue(1, t0 + 1)
            wait_l(0)
            process(0)
            issue(0, t0 + 2)
            wait_l(1)
            process(1)

        wait_l(0)
        process(0)
        pltpu.sync_copy(acc, out_hbm.at[sid, cid])

    return k(e_t, dst_s)


def _tc_pre_kernel(x_ref, wd_ref, ws_ref, xd_ref, xs_ref):
    xb = x_ref[...]
    xd_ref[...] = jnp.dot(xb, wd_ref[...], preferred_element_type=jnp.float32)
    xs_ref[...] = jnp.dot(xb, ws_ref[...], preferred_element_type=jnp.float32)


def _tc_pre(x, w_d, w_s):
    blk = 2000
    return pl.pallas_call(
        _tc_pre_kernel,
        grid=(N_NODES // blk,),
        in_specs=[
            pl.BlockSpec((blk, NODE_DIM), lambda i: (i, 0)),
            pl.BlockSpec((NODE_DIM, HIDDEN), lambda i: (0, 0)),
            pl.BlockSpec((NODE_DIM, HIDDEN), lambda i: (0, 0)),
        ],
        out_specs=[
            pl.BlockSpec((blk, HIDDEN), lambda i: (i, 0)),
            pl.BlockSpec((blk, HIDDEN), lambda i: (i, 0)),
        ],
        out_shape=[
            jax.ShapeDtypeStruct((N_NODES, HIDDEN), jnp.float32),
            jax.ShapeDtypeStruct((N_NODES, HIDDEN), jnp.float32),
        ],
    )(x, w_d, w_s)


def _tc_edge_kernel(
    g_ref, e_ref, w1e_ref, b1_ref, w2_ref, b2_ref, out_ref, out_t_ref
):
    h = g_ref[...]
    h += jnp.dot(e_ref[...], w1e_ref[...], preferred_element_type=jnp.float32)
    h = jnp.maximum(h + b1_ref[...], 0.0)
    res = jnp.dot(h, w2_ref[...], preferred_element_type=jnp.float32) + b2_ref[...]
    out_ref[...] = res
    out_t_ref[...] = res.T


def _tc_edge(g, e, w1e, b1, w2, b2):
    blk = 2560
    return pl.pallas_call(
        _tc_edge_kernel,
        grid=(N_EDGES // blk,),
        in_specs=[
            pl.BlockSpec((blk, HIDDEN), lambda i: (i, 0)),
            pl.BlockSpec((blk, EDGE_DIM), lambda i: (i, 0)),
            pl.BlockSpec((EDGE_DIM, HIDDEN), lambda i: (0, 0)),
            pl.BlockSpec((1, HIDDEN), lambda i: (0, 0)),
            pl.BlockSpec((HIDDEN, EDGE_DIM), lambda i: (0, 0)),
            pl.BlockSpec((1, EDGE_DIM), lambda i: (0, 0)),
        ],
        out_specs=[
            pl.BlockSpec((blk, EDGE_DIM), lambda i: (i, 0)),
            pl.BlockSpec((EDGE_DIM, blk), lambda i: (0, i)),
        ],
        out_shape=[
            jax.ShapeDtypeStruct((N_EDGES, EDGE_DIM), jnp.float32),
            jax.ShapeDtypeStruct((EDGE_DIM, N_EDGES), jnp.float32),
        ],
    )(g, e, w1e, b1, w2, b2)


def _tc_node_kernel(x_ref, a_ref, w1x_ref, w1a_ref, b1_ref, w2_ref, b2_ref, out_ref):
    a = jnp.sum(a_ref[...], axis=0)
    h = jnp.dot(x_ref[...], w1x_ref[...], preferred_element_type=jnp.float32)
    h += jnp.dot(a, w1a_ref[...], preferred_element_type=jnp.float32)
    h = jnp.maximum(h + b1_ref[...], 0.0)
    out_ref[...] = (
        jnp.dot(h, w2_ref[...], preferred_element_type=jnp.float32) + b2_ref[...]
    )


def _tc_node(x, agg2, w1x, w1a, b1, w2, b2):
    blk = 1280
    return pl.pallas_call(
        _tc_node_kernel,
        grid=(N_NODES_PAD // blk,),
        in_specs=[
            pl.BlockSpec((blk, NODE_DIM), lambda i: (i, 0)),
            pl.BlockSpec((SC_SUBCORES, blk, EDGE_DIM), lambda i: (0, i, 0)),
            pl.BlockSpec((NODE_DIM, HIDDEN), lambda i: (0, 0)),
            pl.BlockSpec((EDGE_DIM, HIDDEN), lambda i: (0, 0)),
            pl.BlockSpec((1, HIDDEN), lambda i: (0, 0)),
            pl.BlockSpec((HIDDEN, NODE_DIM), lambda i: (0, 0)),
            pl.BlockSpec((1, NODE_DIM), lambda i: (0, 0)),
        ],
        out_specs=pl.BlockSpec((blk, NODE_DIM), lambda i: (i, 0)),
        out_shape=jax.ShapeDtypeStruct((N_NODES_PAD, NODE_DIM), jnp.float32),
    )(x, agg2, w1x, w1a, b1, w2, b2)


def kernel(x, edge_index, e, fR_W1, fR_b1, fR_W2, fR_b2, fO_W1, fO_b1, fO_W2, fO_b2):
    ei = edge_index.astype(jnp.int32)
    pad = N_EDGES_PAD - N_EDGES
    src2 = jnp.pad(ei[0], (0, pad))
    dst2 = jnp.pad(ei[1], (0, pad))
    dst2_s = jnp.pad(ei[1], (0, pad), constant_values=N_NODES_PAD - 8)

    xd, xs = _tc_pre(x, fR_W1[:NODE_DIM], fR_W1[NODE_DIM : 2 * NODE_DIM])
    g = _sc_gather(xd, xs, dst2, src2)
    e_new, e_t0 = _tc_edge(
        g, e,
        fR_W1[2 * NODE_DIM :],
        fR_b1.reshape(1, HIDDEN),
        fR_W2,
        fR_b2.reshape(1, EDGE_DIM),
    )
    e_t = jnp.pad(e_t0, ((0, 0), (0, pad)))
    parts = _sc_scatter(e_t, dst2_s).reshape(SC_SUBCORES, N_NODES_PAD, EDGE_DIM)
    x_pad = jnp.pad(x, ((0, N_NODES_PAD - N_NODES), (0, 0)))
    x_new = _tc_node(
        x_pad, parts,
        fO_W1[:NODE_DIM],
        fO_W1[NODE_DIM:],
        fO_b1.reshape(1, HIDDEN),
        fO_W2,
        fO_b2.reshape(1, NODE_DIM),
    )[:N_NODES]
    return (x_new, e_new)
